# baseline (speedup 1.0000x reference)
import functools

import jax
import jax.numpy as jnp
from jax import lax
from jax.experimental import pallas as pl
from jax.experimental.pallas import tpu as pltpu
from jax.experimental.pallas import tpu_sc as plsc

_NC = 2
_NS = 16
_NW = _NC * _NS


def _sc_gather(tokens, s_lo, s_len, W):
    B, S = tokens.shape
    V, D = W.shape
    N = B * s_len
    per_w = N // _NW
    CH = 16
    n_ch = per_w // CH
    NBUF = 4

    mesh = plsc.VectorSubcoreMesh(core_axis_name="c", subcore_axis_name="s")

    @functools.partial(
        pl.kernel,
        mesh=mesh,
        out_type=jax.ShapeDtypeStruct((N, D), jnp.float32),
        scratch_types=[
            pltpu.VMEM((per_w,), jnp.int32),
        ] + [pltpu.VMEM((CH, D), jnp.float32) for _ in range(NBUF)]
          + [pltpu.SemaphoreType.DMA for _ in range(2 * NBUF)],
    )
    def k(tokens_hbm, W_hbm, out_hbm, idx_v, *bufs_sems):
        bufs = bufs_sems[:NBUF]
        gsems = bufs_sems[NBUF:2 * NBUF]
        ssems = bufs_sems[2 * NBUF:]
        wid = lax.axis_index("s") * _NC + lax.axis_index("c")
        base = wid * per_w
        w_per_b = s_len // per_w
        b0 = wid // w_per_b
        o0 = s_lo + (wid % w_per_b) * per_w
        pltpu.sync_copy(tokens_hbm.at[b0, pl.ds(o0, per_w)], idx_v)

        def gather(c):
            return pltpu.make_async_copy(
                W_hbm.at[idx_v.at[pl.ds(c * CH, CH)]],
                bufs[c % NBUF], gsems[c % NBUF]
            )

        def scatter(c):
            return pltpu.make_async_copy(
                bufs[c % NBUF],
                out_hbm.at[pl.ds(base + c * CH, CH)], ssems[c % NBUF]
            )

        depth = min(NBUF - 1, n_ch)
        for c in range(depth):
            gather(c).start()
        for c in range(n_ch):
            gather(c).wait()
            if c + depth < n_ch:
                if c + depth >= NBUF:
                    scatter(c + depth - NBUF).wait()
                gather(c + depth).start()
            scatter(c).start()
        for c in range(max(n_ch - depth - 1, 0), n_ch):
            scatter(c).wait()

    return k(tokens, W)


def _tc_ln_chunk(g, P, gamma, beta, B, S, d_model, h, n_chunks, base=None):
    D = g.shape[1]
    S2 = S // n_chunks
    scale = float(d_model) ** 0.5

    def body(*refs):
        g_ref, p_ref, gm_ref, bt_ref = refs[:4]
        o_ref = refs[-1]
        x = g_ref[...] * scale + p_ref[...]
        mu = jnp.mean(x, axis=-1, keepdims=True)
        var = jnp.mean((x - mu) ** 2, axis=-1, keepdims=True)
        xn = (x - mu) / jnp.sqrt(var + 1e-12)
        o_ref[...] = xn * gm_ref[...] + bt_ref[...]

    in_specs = [
        pl.BlockSpec((S2, D), lambda b: (b, 0)),
        pl.BlockSpec((S2, D), lambda b: (h, 0)),
        pl.BlockSpec((1, D), lambda b: (0, 0)),
        pl.BlockSpec((1, D), lambda b: (0, 0)),
    ]
    args = [g, P, gamma.reshape(1, D), beta.reshape(1, D)]
    io_aliases = {}
    if base is not None:
        in_specs.append(pl.BlockSpec(memory_space=pl.ANY))
        args.append(base)
        io_aliases = {4: 0}
    return pl.pallas_call(
        body,
        grid=(B,),
        in_specs=in_specs,
        out_specs=pl.BlockSpec((S2, D), lambda b: (b * n_chunks + h, 0)),
        out_shape=jax.ShapeDtypeStruct((B * S, D), jnp.float32),
        input_output_aliases=io_aliases,
    )(*args)


def kernel(tokens, att_mask, W, P, gamma, beta):
    B, S = tokens.shape
    D = W.shape[1]
    NCHUNK = 4
    SC = S // NCHUNK
    gs = [_sc_gather(tokens, h * SC, SC, W) for h in range(NCHUNK)]
    out = None
    for h in range(NCHUNK):
        out = _tc_ln_chunk(gs[h], P, gamma, beta, B, S, D, h, NCHUNK, base=out)
    return out.reshape(B, S, D), att_mask


# speedup vs baseline: 1.0747x; 1.0747x over previous
import functools

import jax
import jax.numpy as jnp
from jax import lax
from jax.experimental import pallas as pl
from jax.experimental.pallas import tpu as pltpu
from jax.experimental.pallas import tpu_sc as plsc

_NC = 2
_NS = 16
_NW = _NC * _NS


def _sc_gather(tokens, s_lo, s_len, W):
    B, S = tokens.shape
    V, D = W.shape
    N = B * s_len
    per_w = N // _NW
    CH = 16
    n_ch = per_w // CH
    NBUF = 4

    mesh = plsc.VectorSubcoreMesh(core_axis_name="c", subcore_axis_name="s")

    @functools.partial(
        pl.kernel,
        mesh=mesh,
        out_type=jax.ShapeDtypeStruct((N, D), jnp.float32),
        scratch_types=[
            pltpu.VMEM((per_w,), jnp.int32),
        ] + [pltpu.VMEM((CH, D), jnp.float32) for _ in range(NBUF)]
          + [pltpu.SemaphoreType.DMA for _ in range(2 * NBUF)],
    )
    def k(tokens_hbm, W_hbm, out_hbm, idx_v, *bufs_sems):
        bufs = bufs_sems[:NBUF]
        gsems = bufs_sems[NBUF:2 * NBUF]
        ssems = bufs_sems[2 * NBUF:]
        wid = lax.axis_index("s") * _NC + lax.axis_index("c")
        base = wid * per_w
        w_per_b = s_len // per_w
        b0 = wid // w_per_b
        o0 = s_lo + (wid % w_per_b) * per_w
        pltpu.sync_copy(tokens_hbm.at[b0, pl.ds(o0, per_w)], idx_v)

        def gather(c):
            return pltpu.make_async_copy(
                W_hbm.at[idx_v.at[pl.ds(c * CH, CH)]],
                bufs[c % NBUF], gsems[c % NBUF]
            )

        def scatter(c):
            return pltpu.make_async_copy(
                bufs[c % NBUF],
                out_hbm.at[pl.ds(base + c * CH, CH)], ssems[c % NBUF]
            )

        depth = min(NBUF - 1, n_ch)
        for c in range(depth):
            gather(c).start()
        for c in range(n_ch):
            gather(c).wait()
            if c + depth < n_ch:
                if c + depth >= NBUF:
                    scatter(c + depth - NBUF).wait()
                gather(c + depth).start()
            scatter(c).start()
        for c in range(max(n_ch - depth - 1, 0), n_ch):
            scatter(c).wait()

    return k(tokens, W)


def _tc_ln_half(g, P, gamma, beta, B, S, d_model, h, base=None):
    D = g.shape[1]
    S2 = S // 2
    scale = float(d_model) ** 0.5

    def body(*refs):
        g_ref, p_ref, gm_ref, bt_ref = refs[:4]
        o_ref = refs[-1]
        x = g_ref[...] * scale + p_ref[...]
        mu = jnp.mean(x, axis=-1, keepdims=True)
        var = jnp.mean((x - mu) ** 2, axis=-1, keepdims=True)
        xn = (x - mu) / jnp.sqrt(var + 1e-12)
        o_ref[...] = xn * gm_ref[...] + bt_ref[...]

    in_specs = [
        pl.BlockSpec((S2, D), lambda b: (b, 0)),
        pl.BlockSpec((S2, D), lambda b: (h, 0)),
        pl.BlockSpec((1, D), lambda b: (0, 0)),
        pl.BlockSpec((1, D), lambda b: (0, 0)),
    ]
    args = [g, P, gamma.reshape(1, D), beta.reshape(1, D)]
    io_aliases = {}
    if base is not None:
        in_specs.append(pl.BlockSpec(memory_space=pl.ANY))
        args.append(base)
        io_aliases = {4: 0}
    return pl.pallas_call(
        body,
        grid=(B,),
        in_specs=in_specs,
        out_specs=pl.BlockSpec((S2, D), lambda b: (b * 2 + h, 0)),
        out_shape=jax.ShapeDtypeStruct((B * S, D), jnp.float32),
        input_output_aliases=io_aliases,
    )(*args)


def kernel(tokens, att_mask, W, P, gamma, beta):
    B, S = tokens.shape
    D = W.shape[1]
    S2 = S // 2
    g0 = _sc_gather(tokens, 0, S2, W)
    g1 = _sc_gather(tokens, S2, S2, W)
    o0 = _tc_ln_half(g0, P, gamma, beta, B, S, D, 0)
    out = _tc_ln_half(g1, P, gamma, beta, B, S, D, 1, base=o0)
    return out.reshape(B, S, D), att_mask
